# Initial kernel scaffold; baseline (speedup 1.0000x reference)
import jax
import jax.numpy as jnp
from jax.experimental import pallas as pl


def kernel(vector_sequence, sentence_length, word_sequence, W):
    raise NotImplementedError("write your pallas kernel here")


# speedup vs baseline: 2.4051x; 2.4051x over previous
import functools

import jax
import jax.numpy as jnp
from jax import lax
from jax.experimental import pallas as pl
from jax.experimental.pallas import tpu as pltpu
from jax.experimental.pallas import tpu_sc as plsc

_TBLK = 256
_NUM_WORKERS = 32


def _sc_gather_weights(words2d, table_pad):
    rows, cols = words2d.shape
    rpw = rows // _NUM_WORKERS
    mesh = plsc.VectorSubcoreMesh(core_axis_name="c", subcore_axis_name="s")

    @functools.partial(
        pl.kernel,
        out_type=jax.ShapeDtypeStruct((rows, cols), jnp.float32),
        mesh=mesh,
        scratch_types=[
            pltpu.VMEM((rpw, cols), jnp.int32),
            pltpu.VMEM((rpw, cols), jnp.float32),
            pltpu.SemaphoreType.DMA,
        ],
    )
    def gather_kernel(words_hbm, table_hbm, out_hbm, idx_v, val_v, sem):
        wid = lax.axis_index("s") * 2 + lax.axis_index("c")
        base = wid * rpw
        pltpu.sync_copy(words_hbm.at[pl.ds(base, rpw)], idx_v)
        copies = [
            pltpu.async_copy(table_hbm.at[idx_v.at[j]], val_v.at[j], sem)
            for j in range(rpw)
        ]
        for c in copies:
            c.wait()
        pltpu.sync_copy(val_v, out_hbm.at[pl.ds(base, rpw)])

    return gather_kernel(words2d, table_pad)


def _tc_reduce(v, w_tok3, slen, tblk):
    b_dim, t_dim, d_dim = v.shape
    nt = t_dim // tblk

    def body(len_ref, v_ref, w_ref, y_ref, yh_ref, sh_acc, a_acc):
        b = pl.program_id(0)
        tb = pl.program_id(1)
        sl = len_ref[b]

        @pl.when(tb == 0)
        def _init():
            sh_acc[...] = jnp.zeros_like(sh_acc)
            a_acc[...] = jnp.zeros_like(a_acc)

        t0 = tb * tblk

        @pl.when(t0 < sl)
        def _acc():
            vv = v_ref[0]
            ww = w_ref[0, 0]
            it = t0 + lax.broadcasted_iota(jnp.int32, (1, tblk), 1)
            m = (it < sl).astype(jnp.float32)
            rows = jnp.concatenate([m, ww * m], axis=0)
            sh_acc[...] += jnp.dot(rows, vv, preferred_element_type=jnp.float32)
            a_acc[...] += jnp.dot(m, jnp.abs(vv), preferred_element_type=jnp.float32)

        @pl.when(tb == nt - 1)
        def _fin():
            y_ref[0] = sh_acc[0:1] / a_acc[...]
            yh_ref[0] = sh_acc[1:2]

    def v_map(b, tb, len_ref):
        sl = len_ref[b]
        nblk = lax.div(sl + (tblk - 1), tblk)
        tbc = jnp.minimum(tb, jnp.maximum(nblk - 1, 0))
        return (b, tbc, 0)

    def w_map(b, tb, len_ref):
        sl = len_ref[b]
        nblk = lax.div(sl + (tblk - 1), tblk)
        tbc = jnp.minimum(tb, jnp.maximum(nblk - 1, 0))
        return (b, tbc, 0, 0)

    grid_spec = pltpu.PrefetchScalarGridSpec(
        num_scalar_prefetch=1,
        grid=(b_dim, nt),
        in_specs=[
            pl.BlockSpec((1, tblk, d_dim), v_map),
            pl.BlockSpec((1, 1, 1, tblk), w_map),
        ],
        out_specs=[
            pl.BlockSpec((1, 1, d_dim), lambda b, tb, len_ref: (b, 0, 0)),
            pl.BlockSpec((1, 1, d_dim), lambda b, tb, len_ref: (b, 0, 0)),
        ],
        scratch_shapes=[
            pltpu.VMEM((2, d_dim), jnp.float32),
            pltpu.VMEM((1, d_dim), jnp.float32),
        ],
    )

    return pl.pallas_call(
        body,
        grid_spec=grid_spec,
        out_shape=[
            jax.ShapeDtypeStruct((b_dim, 1, d_dim), jnp.float32),
            jax.ShapeDtypeStruct((b_dim, 1, d_dim), jnp.float32),
        ],
        compiler_params=pltpu.CompilerParams(
            dimension_semantics=("parallel", "arbitrary"),
        ),
    )(slen, v, w_tok3)


def kernel(vector_sequence, sentence_length, word_sequence, W):
    b_dim, t_dim, d_dim = vector_sequence.shape
    vocab = W.shape[0]
    slen = sentence_length.astype(jnp.int32)
    words = word_sequence.astype(jnp.int32).reshape(b_dim * t_dim // 128, 128)
    vpad = ((vocab + 1023) // 1024) * 1024
    table_pad = jnp.pad(W.astype(jnp.float32), (0, vpad - vocab))
    w_tok = _sc_gather_weights(words, table_pad)
    w_tok3 = w_tok.reshape(b_dim, t_dim // _TBLK, 1, _TBLK)
    y, y_hat = _tc_reduce(vector_sequence, w_tok3, slen, _TBLK)
    return (y.reshape(b_dim, d_dim), y_hat.reshape(b_dim, d_dim))


# speedup vs baseline: 4.0374x; 1.6787x over previous
import functools

import jax
import jax.numpy as jnp
from jax import lax
from jax.experimental import pallas as pl
from jax.experimental.pallas import tpu as pltpu
from jax.experimental.pallas import tpu_sc as plsc

_TBLK = 512
_NUM_WORKERS = 32


def _sc_gather_weights(words_flat, table_pad):
    n = words_flat.shape[0]
    per = n // _NUM_WORKERS
    vpad = table_pad.shape[0]
    mesh = plsc.VectorSubcoreMesh(core_axis_name="c", subcore_axis_name="s")

    @functools.partial(
        pl.kernel,
        out_type=jax.ShapeDtypeStruct((n,), jnp.float32),
        mesh=mesh,
        scratch_types=[
            pltpu.VMEM((vpad,), jnp.float32),
            pltpu.VMEM((per,), jnp.int32),
            pltpu.VMEM((per,), jnp.float32),
        ],
        compiler_params=pltpu.CompilerParams(needs_layout_passes=False),
    )
    def gather_kernel(words_hbm, table_hbm, out_hbm, tbl_v, idx_v, val_v):
        wid = lax.axis_index("s") * 2 + lax.axis_index("c")
        base = wid * per
        pltpu.sync_copy(table_hbm, tbl_v)
        pltpu.sync_copy(words_hbm.at[pl.ds(base, per)], idx_v)
        for i in range(per // 16):
            idx = idx_v[pl.ds(i * 16, 16)]
            val_v[pl.ds(i * 16, 16)] = plsc.load_gather(tbl_v, [idx])
        pltpu.sync_copy(val_v, out_hbm.at[pl.ds(base, per)])

    return gather_kernel(words_flat, table_pad)


def _tc_reduce(v, w_tok3, slen, tblk):
    b_dim, t_dim, d_dim = v.shape
    nt = t_dim // tblk

    def body(len_ref, v_ref, w_ref, y_ref, yh_ref, sh_acc, a_acc):
        b = pl.program_id(0)
        tb = pl.program_id(1)
        sl = len_ref[b]

        @pl.when(tb == 0)
        def _init():
            sh_acc[...] = jnp.zeros_like(sh_acc)
            a_acc[...] = jnp.zeros_like(a_acc)

        t0 = tb * tblk

        @pl.when(t0 < sl)
        def _acc():
            vv = v_ref[0]
            ww = w_ref[0, 0]
            it = t0 + lax.broadcasted_iota(jnp.int32, (1, tblk), 1)
            m = (it < sl).astype(jnp.float32)
            rows = jnp.concatenate([m, ww * m], axis=0)
            sh_acc[...] += jnp.dot(rows, vv, preferred_element_type=jnp.float32)
            a_acc[...] += jnp.dot(m, jnp.abs(vv), preferred_element_type=jnp.float32)

        @pl.when(tb == nt - 1)
        def _fin():
            y_ref[0] = sh_acc[0:1] / a_acc[...]
            yh_ref[0] = sh_acc[1:2]

    def v_map(b, tb, len_ref):
        sl = len_ref[b]
        nblk = lax.div(sl + (tblk - 1), tblk)
        tbc = jnp.minimum(tb, jnp.maximum(nblk - 1, 0))
        return (b, tbc, 0)

    def w_map(b, tb, len_ref):
        sl = len_ref[b]
        nblk = lax.div(sl + (tblk - 1), tblk)
        tbc = jnp.minimum(tb, jnp.maximum(nblk - 1, 0))
        return (b, tbc, 0, 0)

    grid_spec = pltpu.PrefetchScalarGridSpec(
        num_scalar_prefetch=1,
        grid=(b_dim, nt),
        in_specs=[
            pl.BlockSpec((1, tblk, d_dim), v_map),
            pl.BlockSpec((1, 1, 1, tblk), w_map),
        ],
        out_specs=[
            pl.BlockSpec((1, 1, d_dim), lambda b, tb, len_ref: (b, 0, 0)),
            pl.BlockSpec((1, 1, d_dim), lambda b, tb, len_ref: (b, 0, 0)),
        ],
        scratch_shapes=[
            pltpu.VMEM((2, d_dim), jnp.float32),
            pltpu.VMEM((1, d_dim), jnp.float32),
        ],
    )

    return pl.pallas_call(
        body,
        grid_spec=grid_spec,
        out_shape=[
            jax.ShapeDtypeStruct((b_dim, 1, d_dim), jnp.float32),
            jax.ShapeDtypeStruct((b_dim, 1, d_dim), jnp.float32),
        ],
        compiler_params=pltpu.CompilerParams(
            dimension_semantics=("parallel", "arbitrary"),
        ),
    )(slen, v, w_tok3)


def kernel(vector_sequence, sentence_length, word_sequence, W):
    b_dim, t_dim, d_dim = vector_sequence.shape
    vocab = W.shape[0]
    slen = sentence_length.astype(jnp.int32)
    words = word_sequence.astype(jnp.int32).reshape(-1)
    vpad = ((vocab + 1023) // 1024) * 1024
    table_pad = jnp.pad(W.astype(jnp.float32), (0, vpad - vocab))
    w_tok = _sc_gather_weights(words, table_pad)
    w_tok3 = w_tok.reshape(b_dim, t_dim // _TBLK, 1, _TBLK)
    y, y_hat = _tc_reduce(vector_sequence, w_tok3, slen, _TBLK)
    return (y.reshape(b_dim, d_dim), y_hat.reshape(b_dim, d_dim))
